# baseline (speedup 1.0000x reference)
import functools

import jax
import jax.numpy as jnp
from jax import lax
from jax.experimental import pallas as pl
from jax.experimental.pallas import tpu as pltpu
from jax.experimental.pallas import tpu_sc as plsc

_NBUF = 16
_CHUNK = 32
_LANES = 16


def _make_sc_gather(n, c, m):
  info = plsc.get_sparse_core_info()
  nw = info.num_cores * info.num_subcores
  rows_per_w = m // nw
  n_chunks = rows_per_w // _CHUNK
  n_groups = n_chunks // _NBUF
  assert m == nw * rows_per_w and rows_per_w == n_chunks * _CHUNK
  assert n_chunks == n_groups * _NBUF

  mesh = plsc.VectorSubcoreMesh(core_axis_name="c", subcore_axis_name="s")

  @functools.partial(
      pl.kernel,
      out_type=jax.ShapeDtypeStruct((m, c), jnp.float32),
      mesh=mesh,
      scratch_types=(
          [pltpu.VMEM((rows_per_w,), jnp.int32)]
          + [pltpu.VMEM((_CHUNK, c), jnp.float32) for _ in range(_NBUF)]
          + [pltpu.SemaphoreType.DMA for _ in range(_NBUF)]
      ),
  )
  def gather_kernel(data_hbm, idx_hbm, out_hbm, idx_v, *bufs_sems):
    bufs = bufs_sems[:_NBUF]
    sems = bufs_sems[_NBUF:]
    wid = lax.axis_index("s") * info.num_cores + lax.axis_index("c")
    base = wid * rows_per_w

    pltpu.sync_copy(idx_hbm.at[pl.ds(base, rows_per_w)], idx_v)

    def shift_body(i, carry):
      sl = pl.ds(i * _LANES, _LANES)
      idx_v[sl] = lax.shift_right_logical(idx_v[sl], 3)
      return carry

    lax.fori_loop(0, rows_per_w // _LANES, shift_body, 0)

    def start(chunk, b):
      pltpu.async_copy(
          data_hbm.at[idx_v.at[pl.ds(chunk * _CHUNK, _CHUNK)]],
          bufs[b],
          sems[b],
      )

    def drain(chunk, b):
      pltpu.make_async_copy(
          data_hbm.at[idx_v.at[pl.ds(chunk * _CHUNK, _CHUNK)]],
          bufs[b],
          sems[b],
      ).wait()

    for b in range(_NBUF):
      start(b, b)

    def group_body(g, carry):
      for b in range(_NBUF):
        chunk = g * _NBUF + b
        drain(chunk, b)
        pltpu.sync_copy(
            bufs[b], out_hbm.at[pl.ds(base + chunk * _CHUNK, _CHUNK)]
        )
        start(chunk + _NBUF, b)
      return carry

    lax.fori_loop(0, n_groups - 1, group_body, 0)

    for b in range(_NBUF):
      chunk = (n_groups - 1) * _NBUF + b
      drain(chunk, b)
      pltpu.sync_copy(
          bufs[b], out_hbm.at[pl.ds(base + chunk * _CHUNK, _CHUNK)]
      )

  return gather_kernel


def kernel(data, child_idx, depth):
  n, c = data.shape
  (m,) = child_idx.shape
  return _make_sc_gather(n, c, m)(data, child_idx)


# speedup vs baseline: 1.0773x; 1.0773x over previous
import functools

import jax
import jax.numpy as jnp
from jax import lax
from jax.experimental import pallas as pl
from jax.experimental.pallas import tpu as pltpu
from jax.experimental.pallas import tpu_sc as plsc

_NBUF = 8
_CHUNK = 64
_LANES = 16


def _make_sc_gather(n, c, m):
  info = plsc.get_sparse_core_info()
  nw = info.num_cores * info.num_subcores
  rows_per_w = m // nw
  n_chunks = rows_per_w // _CHUNK
  n_groups = n_chunks // _NBUF
  assert m == nw * rows_per_w and rows_per_w == n_chunks * _CHUNK
  assert n_chunks == n_groups * _NBUF

  mesh = plsc.VectorSubcoreMesh(core_axis_name="c", subcore_axis_name="s")

  @functools.partial(
      pl.kernel,
      out_type=jax.ShapeDtypeStruct((m, c), jnp.float32),
      mesh=mesh,
      scratch_types=(
          [pltpu.VMEM((rows_per_w,), jnp.int32)]
          + [pltpu.VMEM((_CHUNK, c), jnp.float32) for _ in range(_NBUF)]
          + [pltpu.SemaphoreType.DMA for _ in range(_NBUF)]
      ),
  )
  def gather_kernel(data_hbm, idx_hbm, out_hbm, idx_v, *bufs_sems):
    bufs = bufs_sems[:_NBUF]
    sems = bufs_sems[_NBUF:]
    wid = lax.axis_index("s") * info.num_cores + lax.axis_index("c")
    base = wid * rows_per_w

    pltpu.sync_copy(idx_hbm.at[pl.ds(base, rows_per_w)], idx_v)

    def shift_chunk(chunk):
      def shift_body(i, carry):
        sl = pl.ds(chunk * _CHUNK + i * _LANES, _LANES)
        idx_v[sl] = lax.shift_right_logical(idx_v[sl], 3)
        return carry

      lax.fori_loop(0, _CHUNK // _LANES, shift_body, 0)

    def start(chunk, b):
      pltpu.async_copy(
          data_hbm.at[idx_v.at[pl.ds(chunk * _CHUNK, _CHUNK)]],
          bufs[b],
          sems[b],
      )

    def drain(chunk, b):
      pltpu.make_async_copy(
          data_hbm.at[idx_v.at[pl.ds(chunk * _CHUNK, _CHUNK)]],
          bufs[b],
          sems[b],
      ).wait()

    for b in range(_NBUF):
      shift_chunk(b)
      start(b, b)

    def group_body(g, carry):
      for b in range(_NBUF):
        chunk = g * _NBUF + b
        shift_chunk(chunk + _NBUF)
        drain(chunk, b)
        pltpu.sync_copy(
            bufs[b], out_hbm.at[pl.ds(base + chunk * _CHUNK, _CHUNK)]
        )
        start(chunk + _NBUF, b)
      return carry

    lax.fori_loop(0, n_groups - 1, group_body, 0)

    for b in range(_NBUF):
      chunk = (n_groups - 1) * _NBUF + b
      drain(chunk, b)
      pltpu.sync_copy(
          bufs[b], out_hbm.at[pl.ds(base + chunk * _CHUNK, _CHUNK)]
      )

  return gather_kernel


def kernel(data, child_idx, depth):
  n, c = data.shape
  (m,) = child_idx.shape
  return _make_sc_gather(n, c, m)(data, child_idx)
